# baseline (speedup 1.0000x reference)
import functools

import jax
import jax.numpy as jnp
from jax import lax
from jax.experimental import pallas as pl
from jax.experimental.pallas import tpu as pltpu
from jax.experimental.pallas import tpu_sc as plsc

_D = 64
_V = 7
_NW = 32
_CH = 512
_NBUF = 3
_B_SC = 10240
_TB = 128


def _emb_call(idx, tflat, n):
    per_w = n // _NW
    n_chunks = per_w // _CH
    mesh = plsc.VectorSubcoreMesh(core_axis_name="c", subcore_axis_name="s")

    @functools.partial(
        pl.kernel,
        mesh=mesh,
        compiler_params=pltpu.CompilerParams(use_tc_tiling_on_sc=False),
        out_type=jax.ShapeDtypeStruct((n * _D,), jnp.float32),
        scratch_types=(
            [pltpu.VMEM((_V * _D,), jnp.float32)]
            + [pltpu.VMEM((_CH,), jnp.int32) for _ in range(_NBUF)]
            + [pltpu.VMEM((_CH * _D,), jnp.float32) for _ in range(_NBUF)]
            + [pltpu.SemaphoreType.DMA for _ in range(2 * _NBUF)]
        ),
    )
    def _emb(table_hbm, idx_hbm, out_hbm, tab_v, *bufs):
        idx_v = bufs[:_NBUF]
        rows_v = bufs[_NBUF:2 * _NBUF]
        isem = bufs[2 * _NBUF:3 * _NBUF]
        osem = bufs[3 * _NBUF:4 * _NBUF]
        wid = lax.axis_index("s") * 2 + lax.axis_index("c")
        base = wid * per_w

        def start_idx(g, b):
            pltpu.async_copy(idx_hbm.at[pl.ds(base + g * _CH, _CH)],
                             idx_v[b], isem[b])

        for b in range(_NBUF):
            start_idx(b, b)
        pltpu.sync_copy(table_hbm, tab_v)

        def compute(b):
            @plsc.parallel_loop(0, _CH // 16, 1, unroll=2)
            def grp_body(gi):
                ivec = idx_v[b][pl.ds(gi * 16, 16)] * _D
                o0 = gi * (16 * _D)
                for j in range(16):
                    tb = ivec[j]
                    o = o0 + j * _D
                    for c in range(_D // 16):
                        rows_v[b][pl.ds(o + 16 * c, 16)] = (
                            tab_v[pl.ds(tb + 16 * c, 16)])

        def body(g, carry):
            for b in range(_NBUF):
                @pl.when(g % _NBUF == b)
                def _():
                    off = (base + g * _CH) * _D
                    pltpu.make_async_copy(
                        idx_hbm.at[pl.ds(base + g * _CH, _CH)],
                        idx_v[b], isem[b]).wait()
                    @pl.when(g >= _NBUF)
                    def _():
                        poff = (base + (g - _NBUF) * _CH) * _D
                        pltpu.make_async_copy(
                            rows_v[b],
                            out_hbm.at[pl.ds(poff, _CH * _D)],
                            osem[b]).wait()
                    compute(b)
                    pltpu.async_copy(rows_v[b],
                                     out_hbm.at[pl.ds(off, _CH * _D)],
                                     osem[b])
                    @pl.when(g + _NBUF < n_chunks)
                    def _():
                        start_idx(g + _NBUF, b)
            return carry

        lax.fori_loop(0, n_chunks, body, 0)

        for b in range(_NBUF):
            g_last = n_chunks - _NBUF + b
            off = (base + g_last * _CH) * _D
            pltpu.make_async_copy(rows_v[g_last % _NBUF],
                                  out_hbm.at[pl.ds(off, _CH * _D)],
                                  osem[g_last % _NBUF]).wait()

    return _emb(tflat, idx)


def _tc_body(idx_ref, tab_ref, out_ref):
    idx = idx_ref[...]
    tab = tab_ref[...]
    shp = idx.shape + (_D,)
    idx3 = jax.lax.broadcast_in_dim(idx, shp, (0, 1))
    res = jax.lax.broadcast_in_dim(tab[0], shp, (2,))
    for k in range(1, _V):
        res = jnp.where(idx3 == k,
                        jax.lax.broadcast_in_dim(tab[k], shp, (2,)), res)
    out_ref[...] = res


def _tc_call(idx, table, b, l):
    tab8 = jnp.pad(table, ((0, 1), (0, 0)))
    out = pl.pallas_call(
        _tc_body,
        grid=(b // _TB,),
        in_specs=[
            pl.BlockSpec((_TB, l), lambda i: (i, 0)),
            pl.BlockSpec((_V + 1, _D), lambda i: (0, 0)),
        ],
        out_specs=pl.BlockSpec((_TB, l, _D), lambda i: (i, 0, 0)),
        out_shape=jax.ShapeDtypeStruct((b, l, _D), jnp.float32),
    )(idx, tab8)
    return out.reshape(b, l * _D)


def kernel(inputs, table):
    b, l = inputs.shape
    idx = inputs.astype(jnp.int32)
    n_sc = _B_SC * l
    out_sc = _emb_call(idx[:_B_SC].reshape(n_sc), table.reshape(_V * _D),
                       n_sc).reshape(_B_SC, l * _D)
    out_tc = _tc_call(idx[_B_SC:], table, b - _B_SC, l)
    return jnp.concatenate([out_sc, out_tc], axis=0)
